# baseline (speedup 1.0000x reference)
import functools

import jax
import jax.numpy as jnp
from jax import lax
from jax.experimental import pallas as pl
from jax.experimental.pallas import tpu as pltpu
from jax.experimental.pallas import tpu_sc as plsc

N = 16384
DIM = 64
SEL = 2
OUT_W = DIM + SEL
PAIR_W = 2 * DIM

NC = 2
NS = 16
L = 16
NW = NC * NS
B_PER_W = N // NW

TC_ROWS = 1024


def _make_sc_gather():
    mesh = plsc.VectorSubcoreMesh(core_axis_name="c", subcore_axis_name="s")

    @functools.partial(
        pl.kernel,
        mesh=mesh,
        out_type=jax.ShapeDtypeStruct((NW, B_PER_W, PAIR_W), jnp.float32),
        compiler_params=pltpu.CompilerParams(needs_layout_passes=False),
        scratch_types=[
            pltpu.VMEM((B_PER_W,), jnp.int32),
            pltpu.VMEM((B_PER_W,), jnp.int32),
            pltpu.VMEM((B_PER_W, PAIR_W), jnp.float32),
            pltpu.SemaphoreType.DMA,
        ],
    )
    def k(vocab_hbm, table2_hbm, out_hbm, idx_v, pair_v, rows_v, sem):
        cid = lax.axis_index("c")
        scid = lax.axis_index("s")
        wid = scid * NC + cid
        base = wid * B_PER_W

        pltpu.sync_copy(vocab_hbm.at[pl.ds(base, B_PER_W)], idx_v)

        def pair_body(i, _):
            v = idx_v[pl.ds(i * L, L)]
            pair_v[pl.ds(i * L, L)] = lax.shift_right_logical(v, 1)
            return 0

        lax.fori_loop(0, B_PER_W // L, pair_body, 0)

        pltpu.async_copy(table2_hbm.at[pair_v], rows_v, sem).wait()
        pltpu.sync_copy(rows_v, out_hbm.at[wid])

    return k


def _tc_finish(pairs_ref, vid_ref, sid_ref, st_ref, o_ref):
    pairs = pairs_ref[...]
    odd = (vid_ref[...] % 2) == 1
    emb = jnp.where(odd, pairs[:, DIM:], pairs[:, :DIM])
    pick0 = sid_ref[...] == 0
    selrow = jnp.where(pick0, st_ref[0:1, :], st_ref[1:2, :])
    o_ref[...] = jnp.concatenate([emb, selrow], axis=1)


def _make_tc_finish():
    return pl.pallas_call(
        _tc_finish,
        grid=(N // TC_ROWS,),
        in_specs=[
            pl.BlockSpec((TC_ROWS, PAIR_W), lambda i: (i, 0)),
            pl.BlockSpec((TC_ROWS, 1), lambda i: (i, 0)),
            pl.BlockSpec((TC_ROWS, 1), lambda i: (i, 0)),
            pl.BlockSpec((2, SEL), lambda i: (0, 0)),
        ],
        out_specs=pl.BlockSpec((TC_ROWS, OUT_W), lambda i: (i, 0)),
        out_shape=jax.ShapeDtypeStruct((N, OUT_W), jnp.float32),
    )


@jax.jit
def kernel(vocab_ids, selector_ids, table, selector_table):
    vocab_ids = vocab_ids.astype(jnp.int32)
    selector_ids = selector_ids.astype(jnp.int32)
    table2 = table.reshape(table.shape[0] // 2, PAIR_W)
    pairs = _make_sc_gather()(vocab_ids, table2)
    return _make_tc_finish()(pairs.reshape(N, PAIR_W),
                             vocab_ids.reshape(N, 1),
                             selector_ids.reshape(N, 1),
                             selector_table.astype(jnp.float32))
